# baseline (speedup 1.0000x reference)
import functools

import jax
import jax.numpy as jnp
from jax.experimental import pallas as pl
from jax.experimental.pallas import tpu as pltpu

_INT_MIN = -2147483648


def _fused_kernel(ks, fracs, nchunks, rows, nsteps,
                  x_hbm, eps_ref, o_ref, buf_ref, m_ref, sem):
    b = pl.program_id(0)
    r = pl.program_id(1)
    g = b * nchunks + r

    half = rows // 2

    def start_copy(step):
        b1 = step // nchunks
        r1 = step % nchunks
        for h in range(2):
            pltpu.make_async_copy(
                x_hbm.at[b1, pl.ds(r1 * rows + h * half, half)],
                buf_ref.at[step % 2, pl.ds(h * half, half)],
                sem.at[step % 2, h],
            ).start()

    @pl.when(g == 0)
    def _warmup():
        start_copy(0)

    @pl.when(g + 1 < nsteps)
    def _prefetch():
        start_copy(g + 1)

    for h in range(2):
        pltpu.make_async_copy(
            x_hbm.at[b, pl.ds(r * rows + h * half, half)],
            buf_ref.at[g % 2, pl.ds(h * half, half)],
            sem.at[g % 2, h],
        ).wait()

    m_ref[b, pl.ds(r * rows, rows), :] = jnp.mean(buf_ref[g % 2], axis=-1)

    @pl.when(g == nsteps - 1)
    def _select_and_normalize():
        _finish(ks, fracs, m_ref, eps_ref, o_ref)


def _finish(ks, fracs, m_ref, eps_ref, o_ref):
    x = m_ref[:, :, :]
    i = jax.lax.bitcast_convert_type(x, jnp.int32)
    key = jnp.where(i >= 0, i, jnp.int32(_INT_MIN) - i)
    B = x.shape[0]

    def count_lt(t):
        return jnp.sum((key < t).astype(jnp.int32), axis=(1, 2),
                       keepdims=True)

    ks_c = tuple(jnp.int32(k) for k in ks)

    zero = jnp.zeros((B, 1, 1), jnp.int32)
    c0 = count_lt(zero)
    ps = tuple(jnp.where(c0 <= k, zero, zero + jnp.int32(_INT_MIN))
               for k in ks_c)

    def step(j, ps):
        one = jnp.int32(1) << (jnp.int32(30) - j)
        return tuple(
            jnp.where(count_lt(p + one) <= k, p + one, p)
            for p, k in zip(ps, ks_c))

    for j in range(31):
        ps = step(jnp.int32(j), ps)

    vals = [jax.lax.bitcast_convert_type(
                jnp.where(p >= 0, p, jnp.int32(_INT_MIN) - p), jnp.float32)
            for p in ps]

    lof, hif = fracs
    lower = vals[0] * (1.0 - lof) + vals[1] * lof
    upper = vals[2] * (1.0 - hif) + vals[3] * hif
    rng = jnp.maximum(upper - lower, eps_ref[0])
    o_ref[:, :, :] = jnp.clip((x - lower) / rng, 0.0, 1.0)


def kernel(inputs, eps):
    B, H, W, C = inputs.shape
    N = H * W

    def qidx(q):
        pos = q * (N - 1)
        lo = int(pos)
        hi = min(lo + 1, N - 1)
        frac = pos - lo
        return lo, hi, frac

    lo0, lo1, lof = qidx(10.0 / 100.0)
    hi0, hi1, hif = qidx(90.0 / 100.0)
    ks = (lo0, lo1, hi0, hi1)

    R = 112
    nchunks = H // R
    nsteps = B * nchunks
    out = pl.pallas_call(
        functools.partial(_fused_kernel, ks, (lof, hif), nchunks, R, nsteps),
        grid=(B, nchunks),
        in_specs=[
            pl.BlockSpec(memory_space=pl.ANY),
            pl.BlockSpec(memory_space=pltpu.SMEM),
        ],
        out_specs=pl.BlockSpec((B, H, W), lambda b, r: (0, 0, 0)),
        out_shape=jax.ShapeDtypeStruct((B, H, W), jnp.float32),
        scratch_shapes=[
            pltpu.VMEM((2, R, W, C), jnp.float32),
            pltpu.VMEM((B, H, W), jnp.float32),
            pltpu.SemaphoreType.DMA((2, 2)),
        ],
        compiler_params=pltpu.CompilerParams(
            dimension_semantics=("arbitrary", "arbitrary")),
    )(inputs, jnp.reshape(eps, (1,)))

    return out.reshape(B, H, W, 1)


# speedup vs baseline: 1.0587x; 1.0587x over previous
import functools

import jax
import jax.numpy as jnp
from jax.experimental import pallas as pl
from jax.experimental.pallas import tpu as pltpu

_INT_MIN = -2147483648


def _fused_kernel(ks, fracs, nchunks, rows, nsteps,
                  x_hbm, eps_ref, o_ref, buf_ref, m_ref, sem):
    b = pl.program_id(0)
    r = pl.program_id(1)
    g = b * nchunks + r

    def start_copy(step):
        b1 = step // nchunks
        r1 = step % nchunks
        pltpu.make_async_copy(
            x_hbm.at[b1, pl.ds(r1 * rows, rows)],
            buf_ref.at[step % 2],
            sem.at[step % 2],
        ).start()

    @pl.when(g == 0)
    def _warmup():
        start_copy(0)

    @pl.when(g + 1 < nsteps)
    def _prefetch():
        start_copy(g + 1)

    pltpu.make_async_copy(
        x_hbm.at[b, pl.ds(r * rows, rows)],
        buf_ref.at[g % 2],
        sem.at[g % 2],
    ).wait()

    m_ref[b, pl.ds(r * rows, rows), :] = jnp.mean(buf_ref[g % 2], axis=-1)

    @pl.when(g == nsteps - 1)
    def _select_and_normalize():
        _finish(ks, fracs, m_ref, eps_ref, o_ref)


def _finish(ks, fracs, m_ref, eps_ref, o_ref):
    x = m_ref[:, :, :]
    i = jax.lax.bitcast_convert_type(x, jnp.int32)
    key = jnp.where(i >= 0, i, jnp.int32(_INT_MIN) - i)
    B = x.shape[0]

    def count_lt(t):
        return jnp.sum((key < t).astype(jnp.int32), axis=(1, 2),
                       keepdims=True)

    ks_c = (jnp.int32(ks[0]), jnp.int32(ks[2]))

    zero = jnp.zeros((B, 1, 1), jnp.int32)
    c0 = count_lt(zero)
    ps = tuple(jnp.where(c0 <= k, zero, zero + jnp.int32(_INT_MIN))
               for k in ks_c)

    def step(j, ps):
        one = jnp.int32(1) << (jnp.int32(30) - j)
        return tuple(
            jnp.where(count_lt(p + one) <= k, p + one, p)
            for p, k in zip(ps, ks_c))

    for j in range(31):
        ps = step(jnp.int32(j), ps)

    def next_stat(v, k):
        cnt_le = jnp.sum((key <= v).astype(jnp.int32), axis=(1, 2),
                         keepdims=True)
        nxt = jnp.min(jnp.where(key > v, key, jnp.int32(2147483647)),
                      axis=(1, 2), keepdims=True)
        return jnp.where(cnt_le >= k + 2, v, nxt)

    ps = (ps[0], next_stat(ps[0], ks_c[0]), ps[1], next_stat(ps[1], ks_c[1]))

    vals = [jax.lax.bitcast_convert_type(
                jnp.where(p >= 0, p, jnp.int32(_INT_MIN) - p), jnp.float32)
            for p in ps]

    lof, hif = fracs
    lower = vals[0] * (1.0 - lof) + vals[1] * lof
    upper = vals[2] * (1.0 - hif) + vals[3] * hif
    rng = jnp.maximum(upper - lower, eps_ref[0])
    o_ref[:, :, :] = jnp.clip((x - lower) / rng, 0.0, 1.0)


def kernel(inputs, eps):
    B, H, W, C = inputs.shape
    N = H * W

    def qidx(q):
        pos = q * (N - 1)
        lo = int(pos)
        hi = min(lo + 1, N - 1)
        frac = pos - lo
        return lo, hi, frac

    lo0, lo1, lof = qidx(10.0 / 100.0)
    hi0, hi1, hif = qidx(90.0 / 100.0)
    ks = (lo0, lo1, hi0, hi1)

    R = 112
    nchunks = H // R
    nsteps = B * nchunks
    out = pl.pallas_call(
        functools.partial(_fused_kernel, ks, (lof, hif), nchunks, R, nsteps),
        grid=(B, nchunks),
        in_specs=[
            pl.BlockSpec(memory_space=pl.ANY),
            pl.BlockSpec(memory_space=pltpu.SMEM),
        ],
        out_specs=pl.BlockSpec((B, H, W), lambda b, r: (0, 0, 0)),
        out_shape=jax.ShapeDtypeStruct((B, H, W), jnp.float32),
        scratch_shapes=[
            pltpu.VMEM((2, R, W, C), jnp.float32),
            pltpu.VMEM((B, H, W), jnp.float32),
            pltpu.SemaphoreType.DMA((2,)),
        ],
        compiler_params=pltpu.CompilerParams(
            dimension_semantics=("arbitrary", "arbitrary")),
    )(inputs, jnp.reshape(eps, (1,)))

    return out.reshape(B, H, W, 1)
